# speedup vs baseline: 1.2744x; 1.2744x over previous
import functools

import jax
import jax.numpy as jnp
from jax import lax
from jax.experimental import pallas as pl
from jax.experimental.pallas import tpu as pltpu
from jax.experimental.pallas import tpu_sc as plsc


@functools.partial(jax.jit, static_argnums=(3, 4, 5))
def _embed(x_flat, token_table, pos_table, B, S, E):
    N = B * S
    info = plsc.get_sparse_core_info()
    NC, NS = info.num_cores, info.num_subcores
    NW = NC * NS
    n_per_w = N // NW
    mesh = plsc.VectorSubcoreMesh(core_axis_name="c", subcore_axis_name="s")

    @functools.partial(
        pl.kernel,
        mesh=mesh,
        out_type=jax.ShapeDtypeStruct((N, E), jnp.float32),
        scratch_types=[
            pltpu.VMEM((n_per_w,), jnp.int32),
            pltpu.VMEM((n_per_w, E), jnp.float32),
            pltpu.VMEM((n_per_w, E), jnp.float32),
            pltpu.SemaphoreType.DMA,
        ],
    )
    def k(x_hbm, tok_hbm, pos_hbm, out_hbm, idx_v, rows_v, pos_v, sem):
        wid = lax.axis_index("s") * NC + lax.axis_index("c")
        base = wid * n_per_w
        pltpu.sync_copy(x_hbm.at[pl.ds(base, n_per_w)], idx_v)
        gather = pltpu.async_copy(tok_hbm.at[idx_v], rows_v, sem)
        pos_base = lax.rem(base, S)
        pltpu.sync_copy(pos_hbm.at[pl.ds(pos_base, n_per_w)], pos_v)
        gather.wait()

        def row_add(i, carry):
            for j in range(E // 16):
                sl = pl.ds(j * 16, 16)
                rows_v[i, sl] = rows_v[i, sl] + pos_v[i, sl]
            return carry

        lax.fori_loop(0, n_per_w, row_add, 0)
        pltpu.sync_copy(rows_v, out_hbm.at[pl.ds(base, n_per_w)])

    return k(x_flat, token_table, pos_table)


def kernel(x, token_table, pos_table):
    B, S = x.shape
    V, E = token_table.shape
    x_flat = x.reshape(B * S).astype(jnp.int32)
    out = _embed(x_flat, token_table, pos_table, B, S, E)
    return out.reshape(B, S, E)


# speedup vs baseline: 1.3487x; 1.0583x over previous
import functools

import jax
import jax.numpy as jnp
from jax import lax
from jax.experimental import pallas as pl
from jax.experimental.pallas import tpu as pltpu
from jax.experimental.pallas import tpu_sc as plsc


@functools.partial(jax.jit, static_argnums=(3, 4, 5))
def _embed(x_flat, token_table, pos_table, B, S, E):
    N = B * S
    info = plsc.get_sparse_core_info()
    NC, NS = info.num_cores, info.num_subcores
    NW = NC * NS
    n_per_w = N // NW
    mesh = plsc.VectorSubcoreMesh(core_axis_name="c", subcore_axis_name="s")

    @functools.partial(
        pl.kernel,
        mesh=mesh,
        out_type=jax.ShapeDtypeStruct((N, E), jnp.float32),
        scratch_types=[
            pltpu.VMEM((n_per_w,), jnp.int32),
            pltpu.VMEM((n_per_w, E), jnp.float32),
            pltpu.SemaphoreType.DMA,
        ],
    )
    def k(x_hbm, tok_hbm, pos_hbm, out_hbm, idx_v, rows_v, sem):
        wid = lax.axis_index("s") * NC + lax.axis_index("c")
        base = wid * n_per_w
        pltpu.sync_copy(x_hbm.at[pl.ds(base, n_per_w)], idx_v)
        pos_base = lax.rem(base, S)
        pltpu.sync_copy(pos_hbm.at[pl.ds(pos_base, n_per_w)], rows_v)
        pltpu.async_copy(tok_hbm.at[idx_v], rows_v, sem, add=True).wait()
        pltpu.sync_copy(rows_v, out_hbm.at[pl.ds(base, n_per_w)])

    return k(x_flat, token_table, pos_table)


def kernel(x, token_table, pos_table):
    B, S = x.shape
    V, E = token_table.shape
    x_flat = x.reshape(B * S).astype(jnp.int32)
    out = _embed(x_flat, token_table, pos_table, B, S, E)
    return out.reshape(B, S, E)


# speedup vs baseline: 1.3502x; 1.0011x over previous
import functools

import jax
import jax.numpy as jnp
from jax import lax
from jax.experimental import pallas as pl
from jax.experimental.pallas import tpu as pltpu
from jax.experimental.pallas import tpu_sc as plsc


@functools.partial(jax.jit, static_argnums=(3, 4, 5))
def _embed(x_flat, token_table, pos_table, B, S, E):
    N = B * S
    info = plsc.get_sparse_core_info()
    NC, NS = info.num_cores, info.num_subcores
    NW = NC * NS
    n_per_w = N // NW
    CH = 4
    n_chunk = n_per_w // CH
    mesh = plsc.VectorSubcoreMesh(core_axis_name="c", subcore_axis_name="s")

    @functools.partial(
        pl.kernel,
        mesh=mesh,
        out_type=jax.ShapeDtypeStruct((N, E), jnp.float32),
        scratch_types=[
            pltpu.VMEM((n_per_w,), jnp.int32),
            pltpu.VMEM((CH, n_chunk, E), jnp.float32),
            pltpu.SemaphoreType.DMA((CH,)),
            pltpu.SemaphoreType.DMA((CH,)),
            pltpu.SemaphoreType.DMA((CH,)),
        ],
    )
    def k(x_hbm, tok_hbm, pos_hbm, out_hbm, idx_v, buf, sp, sg, sw):
        wid = lax.axis_index("s") * NC + lax.axis_index("c")
        base = wid * n_per_w
        pltpu.sync_copy(x_hbm.at[pl.ds(base, n_per_w)], idx_v)
        pos_base = lax.rem(base, S)
        pd = [
            pltpu.async_copy(
                pos_hbm.at[pl.ds(pos_base + c * n_chunk, n_chunk)],
                buf.at[c], sp.at[c])
            for c in range(CH)
        ]
        gd = []
        for c in range(CH):
            pd[c].wait()
            gd.append(pltpu.async_copy(
                tok_hbm.at[idx_v.at[pl.ds(c * n_chunk, n_chunk)]],
                buf.at[c], sg.at[c], add=True))
        wd = []
        for c in range(CH):
            gd[c].wait()
            wd.append(pltpu.async_copy(
                buf.at[c], out_hbm.at[pl.ds(base + c * n_chunk, n_chunk)],
                sw.at[c]))
        for c in range(CH):
            wd[c].wait()

    return k(x_flat, token_table, pos_table)


def kernel(x, token_table, pos_table):
    B, S = x.shape
    V, E = token_table.shape
    x_flat = x.reshape(B * S).astype(jnp.int32)
    out = _embed(x_flat, token_table, pos_table, B, S, E)
    return out.reshape(B, S, E)


# speedup vs baseline: 1.3762x; 1.0193x over previous
import functools

import jax
import jax.numpy as jnp
from jax import lax
from jax.experimental import pallas as pl
from jax.experimental.pallas import tpu as pltpu
from jax.experimental.pallas import tpu_sc as plsc


@functools.partial(jax.jit, static_argnums=(3, 4, 5))
def _embed(x, token_table, pos_table, B, S, E):
    N = B * S
    info = plsc.get_sparse_core_info()
    NC, NS = info.num_cores, info.num_subcores
    NW = NC * NS
    sp = S // NW
    mesh = plsc.VectorSubcoreMesh(core_axis_name="c", subcore_axis_name="s")

    @functools.partial(
        pl.kernel,
        mesh=mesh,
        out_type=jax.ShapeDtypeStruct((N, E), jnp.float32),
        scratch_types=[
            pltpu.VMEM((B * sp,), jnp.int32),
            pltpu.VMEM((sp, E), jnp.float32),
            pltpu.VMEM((B, sp, E), jnp.float32),
            pltpu.SemaphoreType.DMA((B,)),
            pltpu.SemaphoreType.DMA,
            pltpu.SemaphoreType.DMA((B,)),
            pltpu.SemaphoreType.DMA((B,)),
        ],
    )
    def k(x_hbm, tok_hbm, pos_hbm, out_hbm, idx_v, pos_v, buf, si, so, sg, sw):
        wid = lax.axis_index("s") * NC + lax.axis_index("c")
        p0 = wid * sp
        di = [
            pltpu.async_copy(
                x_hbm.at[pl.ds(b * S + p0, sp)],
                idx_v.at[pl.ds(b * sp, sp)], si.at[b])
            for b in range(B)
        ]
        dp = pltpu.async_copy(pos_hbm.at[pl.ds(p0, sp)], pos_v, so)
        gd = []
        for b in range(B):
            di[b].wait()
            gd.append(pltpu.async_copy(
                tok_hbm.at[idx_v.at[pl.ds(b * sp, sp)]],
                buf.at[b], sg.at[b]))
        dp.wait()
        wd = []
        for b in range(B):
            gd[b].wait()

            def row_add(i, carry, b=b):
                for h in range(E // 16):
                    sl = pl.ds(h * 16, 16)
                    buf[b, i, sl] = buf[b, i, sl] + pos_v[i, sl]
                return carry

            lax.fori_loop(0, sp, row_add, 0)
            wd.append(pltpu.async_copy(
                buf.at[b], out_hbm.at[pl.ds(b * S + p0, sp)], sw.at[b]))
        for b in range(B):
            wd[b].wait()

    return k(x, token_table, pos_table)


def kernel(x, token_table, pos_table):
    B, S = x.shape
    V, E = token_table.shape
    x_flat = x.reshape(B * S).astype(jnp.int32)
    out = _embed(x_flat, token_table, pos_table, B, S, E)
    return out.reshape(B, S, E)


# speedup vs baseline: 1.3886x; 1.0090x over previous
import functools

import jax
import jax.numpy as jnp
from jax import lax
from jax.experimental import pallas as pl
from jax.experimental.pallas import tpu as pltpu
from jax.experimental.pallas import tpu_sc as plsc


@functools.partial(jax.jit, static_argnums=(3, 4, 5))
def _embed(x, token_table, pos_table, B, S, E):
    N = B * S
    info = plsc.get_sparse_core_info()
    NC, NS = info.num_cores, info.num_subcores
    NW = NC * NS
    sp = S // NW
    mesh = plsc.VectorSubcoreMesh(core_axis_name="c", subcore_axis_name="s")

    @functools.partial(
        pl.kernel,
        mesh=mesh,
        out_type=jax.ShapeDtypeStruct((B, S, E), jnp.float32),
        scratch_types=[
            pltpu.VMEM((B * sp,), jnp.int32),
            pltpu.VMEM((sp, E), jnp.float32),
            pltpu.VMEM((B, sp, E), jnp.float32),
            pltpu.SemaphoreType.DMA((B,)),
            pltpu.SemaphoreType.DMA,
            pltpu.SemaphoreType.DMA((B,)),
            pltpu.SemaphoreType.DMA((B,)),
        ],
    )
    def k(x_hbm, tok_hbm, pos_hbm, out_hbm, idx_v, pos_v, buf, si, so, sg, sw):
        wid = lax.axis_index("s") * NC + lax.axis_index("c")
        p0 = wid * sp
        di = [
            pltpu.async_copy(
                x_hbm.at[b, pl.ds(p0, sp)],
                idx_v.at[pl.ds(b * sp, sp)], si.at[b])
            for b in range(B)
        ]
        dp = pltpu.async_copy(pos_hbm.at[pl.ds(p0, sp)], pos_v, so)
        gd = []
        for b in range(B):
            di[b].wait()
            gd.append(pltpu.async_copy(
                tok_hbm.at[idx_v.at[pl.ds(b * sp, sp)]],
                buf.at[b], sg.at[b]))
        dp.wait()
        wd = []
        for b in range(B):
            gd[b].wait()

            def row_add(i, carry, b=b):
                for h in range(E // 16):
                    sl = pl.ds(h * 16, 16)
                    buf[b, i, sl] = buf[b, i, sl] + pos_v[i, sl]
                return carry

            lax.fori_loop(0, sp, row_add, 0)
            wd.append(pltpu.async_copy(
                buf.at[b], out_hbm.at[b, pl.ds(p0, sp)], sw.at[b]))
        for b in range(B):
            wd[b].wait()

    return k(x, token_table, pos_table)


def kernel(x, token_table, pos_table):
    B, S = x.shape
    V, E = token_table.shape
    return _embed(x.astype(jnp.int32), token_table, pos_table, B, S, E)
